# baseline (speedup 1.0000x reference)
import functools
import math

import jax
import jax.numpy as jnp
from jax import lax
from jax.experimental import pallas as pl
from jax.experimental.pallas import tpu as pltpu
from jax.experimental.pallas import tpu_sc as plsc

NUM_NODES = 100000
H_DIM = 256
CHUNK = 40
NUM_CHUNKS = NUM_NODES // CHUNK
NC = 2
NS = 16
NW = NC * NS
BASE_CHUNKS = NUM_CHUNKS // NW
EXTRA_W = NUM_CHUNKS - BASE_CHUNKS * NW
MAX_CHUNKS = BASE_CHUNKS + 1
NBUF = 8
SBUF = 4
LOOKAHEAD = 4
PERIOD = math.lcm(NBUF, SBUF)
NSUPER = -(-(MAX_CHUNKS + SBUF) // PERIOD)

_mesh = plsc.VectorSubcoreMesh(core_axis_name="c", subcore_axis_name="s")


@functools.partial(
    pl.kernel,
    mesh=_mesh,
    out_type=jax.ShapeDtypeStruct((NUM_NODES, H_DIM), jnp.float32),
    scratch_types=[pltpu.VMEM((MAX_CHUNKS * CHUNK,), jnp.int32)]
    + [pltpu.VMEM((CHUNK, H_DIM), jnp.float32) for _ in range(NBUF)]
    + [pltpu.VMEM_SHARED((NS * SBUF * CHUNK, H_DIM), jnp.float32)]
    + [pltpu.SemaphoreType.DMA for _ in range(NBUF + 2 * SBUF)],
)
def _gather_kernel(idx_hbm, table_hbm, out_hbm, idx_all, *scratch):
    rows = list(scratch[:NBUF])
    spm_all = scratch[NBUF]
    sems = scratch[NBUF + 1:]
    gsem = list(sems[:NBUF])
    psem = list(sems[NBUF:NBUF + SBUF])
    wsem = list(sems[NBUF + SBUF:])

    sid = lax.axis_index("s")
    w = lax.axis_index("s") * NC + lax.axis_index("c")
    lo = BASE_CHUNKS * w + jnp.minimum(w, EXTRA_W)
    n_w = BASE_CHUNKS + jnp.where(w < EXTRA_W, 1, 0)

    base_el = lo * CHUNK
    n_base = BASE_CHUNKS * CHUNK
    pltpu.sync_copy(idx_hbm.at[pl.ds(base_el, n_base)],
                    idx_all.at[pl.ds(0, n_base)])

    @pl.when(w < EXTRA_W)
    def _():
        pltpu.sync_copy(idx_hbm.at[pl.ds(base_el + n_base, CHUNK)],
                        idx_all.at[pl.ds(n_base, CHUNK)])

    def gather_desc(j, b):
        off = pl.multiple_of(j * CHUNK, CHUNK)
        idx_slice = idx_all.at[pl.ds(off, CHUNK)]
        return pltpu.make_async_copy(table_hbm.at[idx_slice], rows[b],
                                     gsem[b])

    def spm_slot(s):
        off = pl.multiple_of((sid * SBUF + s) * CHUNK, CHUNK)
        return spm_all.at[pl.ds(off, CHUNK)]

    def push_desc(b, s):
        return pltpu.make_async_copy(rows[b], spm_slot(s), psem[s])

    def write_desc(j, s):
        dst = out_hbm.at[pl.ds((lo + j) * CHUNK, CHUNK)]
        return pltpu.make_async_copy(spm_slot(s), dst, wsem[s])

    for j in range(LOOKAHEAD):
        gather_desc(j, j % NBUF).start()

    def super_ring(it, carry):
        for p in range(PERIOD):
            jb = it * PERIOD + p
            b = p % NBUF
            s = p % SBUF
            jn = jb + LOOKAHEAD

            @pl.when(jn < n_w)
            def _(j=jn, b=(p + LOOKAHEAD) % NBUF):
                gather_desc(j, b).start()

            @pl.when((jb >= SBUF) & (jb - SBUF < n_w))
            def _(j=jb - SBUF, s=s):
                write_desc(j, s).wait()

            @pl.when(jb < n_w)
            def _(j=jb, b=b, s=s):
                gather_desc(j, b).wait()
                push_desc(b, s).start()

            @pl.when((jb >= 1) & (jb - 1 < n_w))
            def _(j=jb - 1, b=(p - 1) % NBUF, s=(p - 1) % SBUF):
                push_desc(b, s).wait()
                write_desc(j, s).start()

        return carry

    lax.fori_loop(0, NSUPER, super_ring, 0)


def kernel(node_id, weight, incidence_in, incidence_out, table):
    node_id = jnp.squeeze(node_id)
    h = _gather_kernel(node_id, table)
    return (weight, h)
